# Initial kernel scaffold; baseline (speedup 1.0000x reference)
import jax
import jax.numpy as jnp
from jax.experimental import pallas as pl


def kernel(xyz, features, W1, b1):
    raise NotImplementedError("write your pallas kernel here")


# speedup vs baseline: 7.9161x; 7.9161x over previous
import jax
import jax.numpy as jnp
from jax import lax
from jax.experimental import pallas as pl

B = 4
N = 8192
SUB = 64
LANE = 128
S = 1024
NSAMPLE = 32
R2 = 0.1 * 0.1
CIN = 67
COUT = 64
T_S = 128
NEG = -1e30


def _fps_kernel(xyz_ref, cen_ref):
    X = xyz_ref[0, 0]
    Y = xyz_ref[0, 1]
    Z = xyz_ref[0, 2]
    iota_lin = (lax.broadcasted_iota(jnp.int32, (SUB, LANE), 0) * LANE
                + lax.broadcasted_iota(jnp.int32, (SUB, LANE), 1))
    iota_s = lax.broadcasted_iota(jnp.int32, (1, S), 1)

    def body(i, carry):
        dist, far, cax, cay, caz = carry
        oh = (iota_lin == far).astype(jnp.float32)
        cx = jnp.sum(X * oh)
        cy = jnp.sum(Y * oh)
        cz = jnp.sum(Z * oh)
        ohs = (iota_s == i).astype(jnp.float32)
        cax = cax + cx * ohs
        cay = cay + cy * ohs
        caz = caz + cz * ohs
        dx = X - cx
        dy = Y - cy
        dz = Z - cz
        d = dx * dx + dy * dy + dz * dz
        dist = jnp.minimum(dist, d)
        m = jnp.max(dist)
        far2 = jnp.min(jnp.where(dist == m, iota_lin, N)).astype(jnp.int32)
        return dist, far2, cax, cay, caz

    dist0 = jnp.full((SUB, LANE), 1e10, dtype=jnp.float32)
    zrow = jnp.zeros((1, S), dtype=jnp.float32)
    _, _, cax, cay, caz = lax.fori_loop(
        0, S, body, (dist0, jnp.int32(0), zrow, zrow, zrow))
    cen_ref[0, 0:1, :] = cax
    cen_ref[0, 1:2, :] = cay
    cen_ref[0, 2:3, :] = caz


def _group_mlp_kernel(g_ref, cen_ref, w_ref, b_ref, out_ref):
    G = g_ref[0]
    X3 = G[:3]
    W = w_ref[...]
    G16 = G.astype(jnp.bfloat16)
    W16 = W.astype(jnp.bfloat16)
    P = lax.dot_general(W16, G16, (((1,), (0,)), ((), ())),
                        preferred_element_type=jnp.float32) + b_ref[...]
    cen = cen_ref[0]
    cen16 = cen.astype(jnp.bfloat16)
    cxdot = lax.dot_general(cen16, G16[:3], (((0,), (0,)), ((), ())),
                            preferred_element_type=jnp.float32)
    x2 = jnp.sum(X3 * X3, axis=0, keepdims=True)
    c2 = lax.dot_general(cen * cen, jnp.ones((3, 1), jnp.float32),
                         (((0,), (0,)), ((), ())),
                         preferred_element_type=jnp.float32, precision=lax.Precision.HIGHEST)
    sqr = c2 + x2 - 2.0 * cxdot
    inball = sqr <= R2
    lidx = lax.broadcasted_iota(jnp.int32, (T_S, N), 1)
    incl = inball.astype(jnp.int32)
    sh = 1
    while sh < N:
        incl = incl + jnp.where(lidx >= sh, jnp.roll(incl, sh, axis=1), 0)
        sh *= 2
    keep = inball & (incl <= NSAMPLE)
    cols = []
    for o in range(COUT):
        tmp = jnp.where(keep, P[o:o + 1, :], NEG)
        cols.append(jnp.max(tmp, axis=1, keepdims=True))
    M = jnp.concatenate(cols, axis=1)
    q = lax.dot_general(cen16, W16[:, :3], (((0,), (1,)), ((), ())),
                        preferred_element_type=jnp.float32)
    out_ref[0] = jnp.maximum(M - q, 0.0)


def kernel(xyz, features, W1, b1):
    xt = jnp.transpose(xyz, (0, 2, 1))
    xr = xt.reshape(B, 3, SUB, LANE)
    cen = pl.pallas_call(
        _fps_kernel,
        grid=(B,),
        in_specs=[pl.BlockSpec((1, 3, SUB, LANE), lambda b: (b, 0, 0, 0))],
        out_specs=pl.BlockSpec((1, 3, S), lambda b: (b, 0, 0)),
        out_shape=jax.ShapeDtypeStruct((B, 3, S), jnp.float32),
    )(xr)
    G = jnp.concatenate([xt, features], axis=1)
    b2 = b1.reshape(COUT, 1)
    out_st = pl.pallas_call(
        _group_mlp_kernel,
        grid=(B, S // T_S),
        in_specs=[
            pl.BlockSpec((1, CIN, N), lambda b, s: (b, 0, 0)),
            pl.BlockSpec((1, 3, T_S), lambda b, s: (b, 0, s)),
            pl.BlockSpec((COUT, CIN), lambda b, s: (0, 0)),
            pl.BlockSpec((COUT, 1), lambda b, s: (0, 0)),
        ],
        out_specs=pl.BlockSpec((1, T_S, COUT), lambda b, s: (b, s, 0)),
        out_shape=jax.ShapeDtypeStruct((B, S, COUT), jnp.float32),
    )(G, cen, W1, b2)
    new_xyz = jnp.transpose(cen, (0, 2, 1))
    new_features = jnp.transpose(out_st, (0, 2, 1))
    return (new_xyz, new_features)


# speedup vs baseline: 8.4200x; 1.0637x over previous
import jax
import jax.numpy as jnp
from jax import lax
from jax.experimental import pallas as pl

B = 4
N = 8192
SUB = 64
LANE = 128
S = 1024
NSAMPLE = 32
R2 = 0.1 * 0.1
CIN = 67
COUT = 64
T_S = 256
NEG = -1e30


def _fps_kernel(xyz_ref, cen_ref):
    X = xyz_ref[0, 0]
    Y = xyz_ref[0, 1]
    Z = xyz_ref[0, 2]
    iota_lin = (lax.broadcasted_iota(jnp.int32, (SUB, LANE), 0) * LANE
                + lax.broadcasted_iota(jnp.int32, (SUB, LANE), 1))
    iota_s = lax.broadcasted_iota(jnp.int32, (1, S), 1)

    def body(i, carry):
        dist, far, cax, cay, caz = carry
        oh = (iota_lin == far).astype(jnp.float32)
        cx = jnp.sum(X * oh)
        cy = jnp.sum(Y * oh)
        cz = jnp.sum(Z * oh)
        ohs = (iota_s == i).astype(jnp.float32)
        cax = cax + cx * ohs
        cay = cay + cy * ohs
        caz = caz + cz * ohs
        dx = X - cx
        dy = Y - cy
        dz = Z - cz
        d = dx * dx + dy * dy + dz * dz
        dist = jnp.minimum(dist, d)
        m = jnp.max(dist)
        far2 = jnp.min(jnp.where(dist == m, iota_lin, N)).astype(jnp.int32)
        return dist, far2, cax, cay, caz

    dist0 = jnp.full((SUB, LANE), 1e10, dtype=jnp.float32)
    zrow = jnp.zeros((1, S), dtype=jnp.float32)
    _, _, cax, cay, caz = lax.fori_loop(
        0, S, body, (dist0, jnp.int32(0), zrow, zrow, zrow))
    cen_ref[0, 0:1, :] = cax
    cen_ref[0, 1:2, :] = cay
    cen_ref[0, 2:3, :] = caz


def _group_mlp_kernel(g_ref, cen_ref, w_ref, b_ref, out_ref):
    G = g_ref[0]
    X3 = G[:3]
    W = w_ref[...]
    G16 = G.astype(jnp.bfloat16)
    W16 = W.astype(jnp.bfloat16)
    P = lax.dot_general(W16, G16, (((1,), (0,)), ((), ())),
                        preferred_element_type=jnp.float32) + b_ref[...]
    cen = cen_ref[0]
    cen16 = cen.astype(jnp.bfloat16)
    cxdot = lax.dot_general(cen16, G16[:3], (((0,), (0,)), ((), ())),
                            preferred_element_type=jnp.float32)
    x2 = jnp.sum(X3 * X3, axis=0, keepdims=True)
    c2 = lax.dot_general(cen * cen, jnp.ones((3, 1), jnp.float32),
                         (((0,), (0,)), ((), ())),
                         preferred_element_type=jnp.float32, precision=lax.Precision.HIGHEST)
    sqr = c2 + x2 - 2.0 * cxdot
    inball = sqr <= R2
    lidx = lax.broadcasted_iota(jnp.int32, (T_S, N), 1)
    incl = inball.astype(jnp.int32)
    sh = 1
    while sh < N:
        incl = incl + jnp.where(lidx >= sh, jnp.roll(incl, sh, axis=1), 0)
        sh *= 2
    keep = inball & (incl <= NSAMPLE)
    cols = []
    for o in range(COUT):
        tmp = jnp.where(keep, P[o:o + 1, :], NEG)
        cols.append(jnp.max(tmp, axis=1, keepdims=True))
    M = jnp.concatenate(cols, axis=1)
    q = lax.dot_general(cen16, W16[:, :3], (((0,), (1,)), ((), ())),
                        preferred_element_type=jnp.float32)
    out_ref[0] = jnp.maximum(M - q, 0.0)


def kernel(xyz, features, W1, b1):
    xt = jnp.transpose(xyz, (0, 2, 1))
    xr = xt.reshape(B, 3, SUB, LANE)
    cen = pl.pallas_call(
        _fps_kernel,
        grid=(B,),
        in_specs=[pl.BlockSpec((1, 3, SUB, LANE), lambda b: (b, 0, 0, 0))],
        out_specs=pl.BlockSpec((1, 3, S), lambda b: (b, 0, 0)),
        out_shape=jax.ShapeDtypeStruct((B, 3, S), jnp.float32),
    )(xr)
    G = jnp.concatenate([xt, features], axis=1)
    b2 = b1.reshape(COUT, 1)
    out_st = pl.pallas_call(
        _group_mlp_kernel,
        grid=(B, S // T_S),
        in_specs=[
            pl.BlockSpec((1, CIN, N), lambda b, s: (b, 0, 0)),
            pl.BlockSpec((1, 3, T_S), lambda b, s: (b, 0, s)),
            pl.BlockSpec((COUT, CIN), lambda b, s: (0, 0)),
            pl.BlockSpec((COUT, 1), lambda b, s: (0, 0)),
        ],
        out_specs=pl.BlockSpec((1, T_S, COUT), lambda b, s: (b, s, 0)),
        out_shape=jax.ShapeDtypeStruct((B, S, COUT), jnp.float32),
    )(G, cen, W1, b2)
    new_xyz = jnp.transpose(cen, (0, 2, 1))
    new_features = jnp.transpose(out_st, (0, 2, 1))
    return (new_xyz, new_features)
